# Initial kernel scaffold; baseline (speedup 1.0000x reference)
import jax
import jax.numpy as jnp
from jax.experimental import pallas as pl


def kernel(v, e, edge_index, Wq, bq, Wk, bk, Wv, bv, We, be):
    raise NotImplementedError("write your pallas kernel here")


# speedup vs baseline: 2.9359x; 2.9359x over previous
import functools

import numpy as np
import jax
import jax.numpy as jnp
from jax import lax
from jax.experimental import pallas as pl
from jax.experimental.pallas import tpu as pltpu
from jax.experimental.pallas import tpu_sc as plsc

N = 10000
E = 160000
IN_DIM = 256
H = 8
D = 32
HD = H * D

NC, NS = 2, 16
NW = NC * NS
EPW = E // NW
G_CH = 200
S_CH = 640
S_G = 32
S_CB = 640
S_TR = S_CB + 16
NRN = 632
VROWS = (NW // 2) * NRN
DUMMY = NRN
ACC_R = NRN + 8

_SCALE = np.float32(1.0 / np.sqrt(np.float32(D)))


def _mesh():
    return plsc.VectorSubcoreMesh(
        core_axis_name="c", subcore_axis_name="s", num_cores=NC, num_subcores=NS
    )


def _qkv_body(v_ref, w_ref, b_ref, o_ref):
    o_ref[...] = (
        jnp.dot(v_ref[...], w_ref[...], preferred_element_type=jnp.float32)
        + b_ref[...]
    )


def _qkv_call(v, wT, b):
    return pl.pallas_call(
        _qkv_body,
        out_shape=jax.ShapeDtypeStruct((N, 3 * HD), jnp.float32),
    )(v, wT, b)


def _gather_body(kt, qt, vt, srci, dsti, kout, qout, vout, idxs_v, idxd_v,
                 rows_v, sem):
    c = lax.axis_index("c")
    s = lax.axis_index("s")
    wid = s * NC + c
    base = wid * EPW
    pltpu.sync_copy(srci.at[pl.ds(base, EPW)], idxs_v)
    pltpu.sync_copy(dsti.at[pl.ds(base, EPW)], idxd_v)

    def step(i, carry):
        off = i * G_CH
        sl = pl.ds(off, G_CH)
        out_sl = pl.ds(base + off, G_CH)
        pltpu.async_copy(kt.at[idxs_v.at[sl]], rows_v, sem).wait()
        pltpu.sync_copy(rows_v, kout.at[out_sl])
        pltpu.async_copy(qt.at[idxd_v.at[sl]], rows_v, sem).wait()
        pltpu.sync_copy(rows_v, qout.at[out_sl])
        pltpu.async_copy(vt.at[idxs_v.at[sl]], rows_v, sem).wait()
        pltpu.sync_copy(rows_v, vout.at[out_sl])
        return carry

    lax.fori_loop(0, EPW // G_CH, step, 0)


def _gather_call(k_t, q_t, v_t, src, dst):
    f = pl.kernel(
        _gather_body,
        out_type=[jax.ShapeDtypeStruct((E, HD), jnp.float32)] * 3,
        mesh=_mesh(),
        scratch_types=[
            pltpu.VMEM((EPW,), jnp.int32),
            pltpu.VMEM((EPW,), jnp.int32),
            pltpu.VMEM((G_CH, HD), jnp.float32),
            pltpu.SemaphoreType.DMA,
        ],
    )
    return f(k_t, q_t, v_t, src, dst)


def _edge_body(e_ref, ks_ref, qd_ref, vs_ref, weT_ref, be_ref,
               eout_ref, msg_ref, s2_ref):
    p = (
        jnp.dot(e_ref[...], weT_ref[...], preferred_element_type=jnp.float32)
        + be_ref[...]
    )
    score = ks_ref[...] * qd_ref[...] * _SCALE * p
    eout_ref[...] = score
    hsel = (
        lax.broadcasted_iota(jnp.int32, (HD, H), 0) // D
        == lax.broadcasted_iota(jnp.int32, (HD, H), 1)
    ).astype(jnp.float32)
    hselT = (
        lax.broadcasted_iota(jnp.int32, (H, HD), 1) // D
        == lax.broadcasted_iota(jnp.int32, (H, HD), 0)
    ).astype(jnp.float32)
    t = jnp.dot(score, hsel, preferred_element_type=jnp.float32)
    s = jnp.exp(jnp.clip(t, -5.0, 5.0))
    sb = jnp.dot(s, hselT, preferred_element_type=jnp.float32)
    msg = vs_ref[...] * sb
    msg_ref[0] = msg[:, :HD // 2]
    msg_ref[1] = msg[:, HD // 2:]
    s2 = jnp.concatenate([s, s], axis=1)
    rsel = lax.broadcasted_iota(jnp.int32, (EB // 8, EB), 0)
    esel = lax.broadcasted_iota(jnp.int32, (EB // 8, EB), 1)
    blocks = [
        (rsel * 8 + je == esel).astype(jnp.float32) for je in range(8)
    ]
    s2_ref[...] = jnp.concatenate(
        [jnp.dot(b, s2, preferred_element_type=jnp.float32) for b in blocks],
        axis=1)


EB = 1600


def _edge_call(e, ks, qd, vs, weT, be):
    grid = (E // EB,)
    bs_e = pl.BlockSpec((EB, IN_DIM), lambda i: (i, 0))
    bs_hd = pl.BlockSpec((EB, HD), lambda i: (i, 0))
    return pl.pallas_call(
        _edge_body,
        grid=grid,
        in_specs=[
            bs_e, bs_hd, bs_hd, bs_hd,
            pl.BlockSpec((IN_DIM, HD), lambda i: (0, 0)),
            pl.BlockSpec((1, HD), lambda i: (0, 0)),
        ],
        out_specs=[
            bs_hd,
            pl.BlockSpec((2, EB, HD // 2), lambda i: (0, i, 0)),
            pl.BlockSpec((EB // 8, 128), lambda i: (i, 0)),
        ],
        out_shape=[
            jax.ShapeDtypeStruct((E, HD), jnp.float32),
            jax.ShapeDtypeStruct((2, E, HD // 2), jnp.float32),
            jax.ShapeDtypeStruct((E // 8, 128), jnp.float32),
        ],
    )(e, ks, qd, vs, weT, be)


def _vgather(vec, idx):
    dnums = lax.GatherDimensionNumbers(
        offset_dims=(), collapsed_slice_dims=(0,), start_index_map=(0,)
    )
    return lax.gather(
        vec, idx[:, None], dnums, slice_sizes=(1,),
        mode=lax.GatherScatterMode.PROMISE_IN_BOUNDS,
    )


def _scatter_body(m2f, s2p, dsti, vout,
                  idx_v, cbl, cbe, cbm, idx8_v, gbuf, sgbuf, acc, accz, sem):
    c = lax.axis_index("c")
    sid = lax.axis_index("s")
    widx = c * NS + sid
    nr = widx // 2
    ch = widx % 2
    base = nr * NRN

    iota16 = lax.broadcasted_iota(jnp.int32, (16,), 0)
    zeros16 = iota16 * 0
    zf16 = zeros16.astype(jnp.float32)

    def zinit(rr, c2):
        for k in range(8):
            acc[rr, pl.ds(k * 16, 16)] = zf16
        accz[pl.ds(rr * 16, 16)] = zf16
        return c2

    lax.fori_loop(0, ACC_R, zinit, 0)

    def chunk(i, carry):
        off = i * S_CH
        pltpu.sync_copy(dsti.at[pl.ds(off, S_CH)], idx_v)

        def pre(k, c2):
            cbl[pl.ds(k * 16, 16)] = zeros16 + DUMMY
            cbe[pl.ds(k * 16, 16)] = zeros16
            cbm[pl.ds(k * 16, 16)] = zeros16
            return c2

        lax.fori_loop(0, S_TR // 16, pre, 0)

        def scan(g, pos):
            dstv = idx_v[pl.ds(g * 16, 16)]
            l = dstv - base
            inr = (l >= 0) & (l < NRN)
            gid = iota16 + (off + g * 16)
            inri = inr.astype(jnp.int32)
            csum = plsc.cumsum(inri)
            tgt = jnp.where(inr, pos + (csum - inri), S_CB + iota16)
            plsc.store_scatter(cbl, [tgt], jnp.where(inr, l, DUMMY))
            plsc.store_scatter(cbe, [tgt], jnp.where(inr, gid, 0))
            plsc.store_scatter(cbm, [tgt], jnp.where(inr, gid + ch * E, 0))
            return pos + csum[15]

        pos = lax.fori_loop(0, S_CH // 16, scan, 0)

        def rnd(r, c2):
            rb = pl.multiple_of(r * S_G, S_G)
            def i8(g2, c4):
                ev = cbe[pl.ds(rb + g2 * 16, 16)]
                idx8_v[pl.ds(g2 * 16, 16)] = ev // 8
                return c4

            lax.fori_loop(0, S_G // 16, i8, 0)
            pltpu.async_copy(m2f.at[cbm.at[pl.ds(rb, S_G)]], gbuf,
                             sem).wait()
            pltpu.async_copy(s2p.at[idx8_v], sgbuf, sem).wait()

            def one(jg, c3):
                lv = cbl[pl.ds(rb + jg * 16, 16)]
                ev = cbe[pl.ds(rb + jg * 16, 16)]
                for jj in range(16):
                    lrow = lv[jj]
                    j = jg * 16 + jj
                    for k in range(8):
                        plsc.addupdate(acc.at[lrow, pl.ds(k * 16, 16)],
                                       gbuf[j, pl.ds(k * 16, 16)])
                    q = lax.rem(ev[jj], 8)
                    plsc.addupdate(accz.at[pl.ds(lrow * 16, 16)],
                                   sgbuf[j, pl.ds(q * 16, 16)])
                return c3

            lax.fori_loop(0, S_G // 16, one, 0)
            return c2

        lax.fori_loop(0, (pos + S_G - 1) // S_G, rnd, 0)
        return carry

    lax.fori_loop(0, E // S_CH, chunk, 0)

    def norm(rr, c2):
        zrow = accz[pl.ds(rr * 16, 16)]
        for k in range(8):
            hk = 4 * ch + k // 2
            zs = _vgather(zrow, zeros16 + hk)
            vals = acc[rr, pl.ds(k * 16, 16)]
            acc[rr, pl.ds(k * 16, 16)] = vals / (zs + 1e-6)
        return c2

    lax.fori_loop(0, NRN, norm, 0)

    def wb(k, c2):
        pltpu.sync_copy(
            acc.at[pl.ds(k * 8, 8), pl.ds(0, HD // 2)],
            vout.at[pl.ds(base + k * 8, 8), pl.ds(ch * (HD // 2), HD // 2)])
        return c2

    lax.fori_loop(0, NRN // 8, wb, 0)


def _scatter_call(m2, s2, dst):
    m2f = m2.reshape(2 * E, HD // 2)
    f = pl.kernel(
        _scatter_body,
        out_type=jax.ShapeDtypeStruct((VROWS, HD), jnp.float32),
        mesh=_mesh(),
        compiler_params=pltpu.CompilerParams(
            needs_layout_passes=False,
            internal_scratch_in_bytes=1 << 20,
        ),
        scratch_types=[
            pltpu.VMEM((S_CH,), jnp.int32),
            pltpu.VMEM((S_TR,), jnp.int32),
            pltpu.VMEM((S_TR,), jnp.int32),
            pltpu.VMEM((S_TR,), jnp.int32),
            pltpu.VMEM((S_G,), jnp.int32),
            pltpu.VMEM((S_G, HD // 2), jnp.float32),
            pltpu.VMEM((S_G, 128), jnp.float32),
            pltpu.VMEM((ACC_R, HD // 2), jnp.float32),
            pltpu.VMEM((ACC_R * 2 * H,), jnp.float32),
            pltpu.SemaphoreType.DMA,
        ],
    )
    return f(m2f, s2, dst)


@jax.jit
def _run(v, e, edge_index, Wq, bq, Wk, bk, Wv, bv, We, be):
    wT = jnp.concatenate([Wq, Wk, Wv], axis=0).T
    b = jnp.concatenate([bq, bk, bv])[None, :]
    qkv = _qkv_call(v, wT, b)
    q_t = qkv[:, :HD]
    k_t = qkv[:, HD:2 * HD]
    v_t = qkv[:, 2 * HD:]
    src = edge_index[0]
    dst = edge_index[1]
    ks, qd, vs = _gather_call(k_t, q_t, v_t, src, dst)
    e_out, m2, s2 = _edge_call(e, ks, qd, vs, We.T, be[None, :])
    v_pad = _scatter_call(m2, s2, dst)
    return v_pad[:N].reshape(N, H, D), e_out.reshape(E, H, D)


def kernel(v, e, edge_index, Wq, bq, Wk, bk, Wv, bv, We, be):
    return _run(v, e, edge_index, Wq, bq, Wk, bk, Wv, bv, We, be)


# speedup vs baseline: 11.7838x; 4.0137x over previous
import functools

import numpy as np
import jax
import jax.numpy as jnp
from jax import lax
from jax.experimental import pallas as pl
from jax.experimental.pallas import tpu as pltpu
from jax.experimental.pallas import tpu_sc as plsc

N = 10000
E = 160000
IN_DIM = 256
H = 8
D = 32
HD = H * D

NC, NS = 2, 16
NW = NC * NS
EPW = E // NW
G_CH = 200
S_CH = 640
S_G = 64
S_CB = 640
S_TR = S_CB + 16
NRN = 632
VROWS = (NW // 2) * NRN
DUMMY = NRN
ACC_R = NRN + 8

_SCALE = np.float32(1.0 / np.sqrt(np.float32(D)))


def _mesh():
    return plsc.VectorSubcoreMesh(
        core_axis_name="c", subcore_axis_name="s", num_cores=NC, num_subcores=NS
    )


def _qkv_body(v_ref, w_ref, b_ref, o_ref):
    o_ref[...] = (
        jnp.dot(v_ref[...], w_ref[...], preferred_element_type=jnp.float32)
        + b_ref[...]
    )


def _qkv_call(v, wT, b):
    return pl.pallas_call(
        _qkv_body,
        out_shape=jax.ShapeDtypeStruct((N, 3 * HD), jnp.float32),
    )(v, wT, b)


def _gather_body(kt, qt, vt, srci, dsti, kout, qout, vout, idxs_v, idxd_v,
                 rows_v, sem):
    c = lax.axis_index("c")
    s = lax.axis_index("s")
    wid = s * NC + c
    base = wid * EPW
    pltpu.sync_copy(srci.at[pl.ds(base, EPW)], idxs_v)
    pltpu.sync_copy(dsti.at[pl.ds(base, EPW)], idxd_v)

    def step(i, carry):
        off = i * G_CH
        sl = pl.ds(off, G_CH)
        out_sl = pl.ds(base + off, G_CH)
        pltpu.async_copy(kt.at[idxs_v.at[sl]], rows_v, sem).wait()
        pltpu.sync_copy(rows_v, kout.at[out_sl])
        pltpu.async_copy(qt.at[idxd_v.at[sl]], rows_v, sem).wait()
        pltpu.sync_copy(rows_v, qout.at[out_sl])
        pltpu.async_copy(vt.at[idxs_v.at[sl]], rows_v, sem).wait()
        pltpu.sync_copy(rows_v, vout.at[out_sl])
        return carry

    lax.fori_loop(0, EPW // G_CH, step, 0)


def _gather_call(k_t, q_t, v_t, src, dst):
    f = pl.kernel(
        _gather_body,
        out_type=[jax.ShapeDtypeStruct((E, HD), jnp.float32)] * 3,
        mesh=_mesh(),
        scratch_types=[
            pltpu.VMEM((EPW,), jnp.int32),
            pltpu.VMEM((EPW,), jnp.int32),
            pltpu.VMEM((G_CH, HD), jnp.float32),
            pltpu.SemaphoreType.DMA,
        ],
    )
    return f(k_t, q_t, v_t, src, dst)


def _edge_body(e_ref, ks_ref, qd_ref, vs_ref, weT_ref, be_ref,
               eout_ref, msg_ref, s2_ref):
    p = (
        jnp.dot(e_ref[...], weT_ref[...], preferred_element_type=jnp.float32)
        + be_ref[...]
    )
    score = ks_ref[...] * qd_ref[...] * _SCALE * p
    eout_ref[...] = score
    hsel = (
        lax.broadcasted_iota(jnp.int32, (HD, H), 0) // D
        == lax.broadcasted_iota(jnp.int32, (HD, H), 1)
    ).astype(jnp.float32)
    hselT = (
        lax.broadcasted_iota(jnp.int32, (H, HD), 1) // D
        == lax.broadcasted_iota(jnp.int32, (H, HD), 0)
    ).astype(jnp.float32)
    t = jnp.dot(score, hsel, preferred_element_type=jnp.float32)
    s = jnp.exp(jnp.clip(t, -5.0, 5.0))
    sb = jnp.dot(s, hselT, preferred_element_type=jnp.float32)
    msg = vs_ref[...] * sb
    msg_ref[0] = msg[:, :HD // 2]
    msg_ref[1] = msg[:, HD // 2:]
    s2 = jnp.concatenate([s, s], axis=1)
    rsel = lax.broadcasted_iota(jnp.int32, (EB // 8, EB), 0)
    esel = lax.broadcasted_iota(jnp.int32, (EB // 8, EB), 1)
    blocks = [
        (rsel * 8 + je == esel).astype(jnp.float32) for je in range(8)
    ]
    s2_ref[...] = jnp.concatenate(
        [jnp.dot(b, s2, preferred_element_type=jnp.float32) for b in blocks],
        axis=1)


EB = 1600


def _edge_call(e, ks, qd, vs, weT, be):
    grid = (E // EB,)
    bs_e = pl.BlockSpec((EB, IN_DIM), lambda i: (i, 0))
    bs_hd = pl.BlockSpec((EB, HD), lambda i: (i, 0))
    return pl.pallas_call(
        _edge_body,
        grid=grid,
        in_specs=[
            bs_e, bs_hd, bs_hd, bs_hd,
            pl.BlockSpec((IN_DIM, HD), lambda i: (0, 0)),
            pl.BlockSpec((1, HD), lambda i: (0, 0)),
        ],
        out_specs=[
            bs_hd,
            pl.BlockSpec((2, EB, HD // 2), lambda i: (0, i, 0)),
            pl.BlockSpec((EB // 8, 128), lambda i: (i, 0)),
        ],
        out_shape=[
            jax.ShapeDtypeStruct((E, HD), jnp.float32),
            jax.ShapeDtypeStruct((2, E, HD // 2), jnp.float32),
            jax.ShapeDtypeStruct((E // 8, 128), jnp.float32),
        ],
    )(e, ks, qd, vs, weT, be)


def _vgather(vec, idx):
    dnums = lax.GatherDimensionNumbers(
        offset_dims=(), collapsed_slice_dims=(0,), start_index_map=(0,)
    )
    return lax.gather(
        vec, idx[:, None], dnums, slice_sizes=(1,),
        mode=lax.GatherScatterMode.PROMISE_IN_BOUNDS,
    )


def _scatter_body(m2f, s2p, dsti, vout,
                  idx2, cbl2, cbe2, cbm2, idx82, idx8e,
                  gbuf, sgbuf, acc, accz, dsem, gsem):
    c = lax.axis_index("c")
    sid = lax.axis_index("s")
    widx = c * NS + sid
    nr = widx // 2
    ch = widx % 2
    base = nr * NRN
    NCH = E // S_CH

    iota16 = lax.broadcasted_iota(jnp.int32, (16,), 0)
    zeros16 = iota16 * 0
    zf16 = zeros16.astype(jnp.float32)

    def zinit(rr, c2):
        for k in range(8):
            acc[rr, pl.ds(k * 16, 16)] = zf16
        accz[pl.ds(rr * 16, 16)] = zf16
        return c2

    lax.fori_loop(0, ACC_R, zinit, 0)

    def scan(bank, off):
        cb = bank * S_TR
        def pre(k, c2):
            cbl2[pl.ds(cb + k * 16, 16)] = zeros16 + DUMMY
            return c2

        lax.fori_loop(0, S_TR // 16, pre, 0)

        def sgrp(g, pos):
            dstv = idx2[pl.ds(bank * S_CH + g * 16, 16)]
            l = dstv - base
            inr = (l >= 0) & (l < NRN)
            gid = iota16 + (off + g * 16)
            inri = inr.astype(jnp.int32)
            csum = plsc.cumsum(inri)
            tgt = cb + jnp.where(inr, pos + (csum - inri), S_CB + iota16)
            plsc.store_scatter(cbl2, [tgt], jnp.where(inr, l, DUMMY))
            plsc.store_scatter(cbe2, [tgt], jnp.where(inr, gid, 0))
            plsc.store_scatter(cbm2, [tgt], jnp.where(inr, gid + ch * E, 0))
            return pos + csum[15]

        return lax.fori_loop(0, S_CH // 16, sgrp, 0)

    def fire(bank):
        cb = bank * S_TR

        def i8(g, c2):
            ev = cbe2[pl.ds(cb + g * 16, 16)]
            idx82[pl.ds(bank * S_G + g * 16, 16)] = ev // 8
            return c2

        lax.fori_loop(0, S_G // 16, i8, 0)
        pltpu.async_copy(m2f.at[cbm2.at[pl.ds(cb, S_G)]], gbuf, gsem)
        pltpu.async_copy(s2p.at[idx82.at[pl.ds(bank * S_G, S_G)]], sgbuf,
                         gsem)

    def drain_gathers():
        pltpu.make_async_copy(m2f.at[pl.ds(0, S_G)], gbuf, gsem).wait()
        pltpu.make_async_copy(m2f.at[pl.ds(0, S_G)], sgbuf, gsem).wait()

    def process_block(bank, rb):
        cb = bank * S_TR

        def one(jg, c3):
            lv = cbl2[pl.ds(cb + rb + jg * 16, 16)]
            ev = cbe2[pl.ds(cb + rb + jg * 16, 16)]
            for jj in range(16):
                lrow = lv[jj]
                j = jg * 16 + jj
                for k in range(8):
                    plsc.addupdate(acc.at[lrow, pl.ds(k * 16, 16)],
                                   gbuf[j, pl.ds(k * 16, 16)])
                q = lax.rem(ev[jj], 8)
                plsc.addupdate(accz.at[pl.ds(lrow * 16, 16)],
                               sgbuf[j, pl.ds(q * 16, 16)])
            return c3

        lax.fori_loop(0, S_G // 16, one, 0)

    def extra_rounds(bank, pos):
        cb = bank * S_TR
        nrnd = (pos + S_G - 1) // S_G

        @pl.when(nrnd > 1)
        def _():
            def rnd(r, c2):
                rb = pl.multiple_of(r * S_G, S_G)

                def i8(g, c3):
                    ev = cbe2[pl.ds(cb + rb + g * 16, 16)]
                    idx8e[pl.ds(g * 16, 16)] = ev // 8
                    return c3

                lax.fori_loop(0, S_G // 16, i8, 0)
                pltpu.async_copy(
                    m2f.at[cbm2.at[pl.ds(cb + rb, S_G)]], gbuf, gsem).wait()
                pltpu.async_copy(s2p.at[idx8e], sgbuf, gsem).wait()
                process_block(bank, rb)
                return c2

            lax.fori_loop(1, nrnd, rnd, 0)

    pltpu.sync_copy(dsti.at[pl.ds(0, S_CH)], idx2.at[pl.ds(0, S_CH)])
    pos0 = scan(0, 0)
    fire(0)
    pltpu.async_copy(dsti.at[pl.ds(S_CH, S_CH)], idx2.at[pl.ds(S_CH, S_CH)],
                     dsem)

    def step(i, pos_i):
        bank = lax.rem(i, 2)
        nbank = 1 - bank
        pltpu.make_async_copy(dsti.at[pl.ds(0, S_CH)],
                              idx2.at[pl.ds(nbank * S_CH, S_CH)], dsem).wait()
        nxt = jnp.minimum(i + 1, NCH - 1)
        pos_n = scan(nbank, nxt * S_CH)

        @pl.when(i < NCH - 1)
        def _():
            nxt2 = jnp.minimum(i + 2, NCH - 1)
            pltpu.async_copy(dsti.at[pl.ds(nxt2 * S_CH, S_CH)],
                             idx2.at[pl.ds(bank * S_CH, S_CH)], dsem)

        drain_gathers()
        process_block(bank, 0)
        extra_rounds(bank, pos_i)

        @pl.when(i + 1 < NCH)
        def _():
            fire(nbank)

        return pos_n

    lax.fori_loop(0, NCH, step, pos0)

    def norm(rr, c2):
        zrow = accz[pl.ds(rr * 16, 16)]
        for k in range(8):
            hk = 4 * ch + k // 2
            zs = _vgather(zrow, zeros16 + hk)
            vals = acc[rr, pl.ds(k * 16, 16)]
            acc[rr, pl.ds(k * 16, 16)] = vals / (zs + 1e-6)
        return c2

    lax.fori_loop(0, NRN, norm, 0)

    def wb(k, c2):
        pltpu.sync_copy(
            acc.at[pl.ds(k * 8, 8), pl.ds(0, HD // 2)],
            vout.at[pl.ds(base + k * 8, 8), pl.ds(ch * (HD // 2), HD // 2)])
        return c2

    lax.fori_loop(0, NRN // 8, wb, 0)


def _scatter_call(m2, s2, dst):
    m2f = m2.reshape(2 * E, HD // 2)
    f = pl.kernel(
        _scatter_body,
        out_type=jax.ShapeDtypeStruct((VROWS, HD), jnp.float32),
        mesh=_mesh(),
        compiler_params=pltpu.CompilerParams(
            needs_layout_passes=False,
            internal_scratch_in_bytes=1 << 20,
        ),
        scratch_types=[
            pltpu.VMEM((2 * S_CH,), jnp.int32),
            pltpu.VMEM((2 * S_TR,), jnp.int32),
            pltpu.VMEM((2 * S_TR,), jnp.int32),
            pltpu.VMEM((2 * S_TR,), jnp.int32),
            pltpu.VMEM((2 * S_G,), jnp.int32),
            pltpu.VMEM((S_G,), jnp.int32),
            pltpu.VMEM((S_G, HD // 2), jnp.float32),
            pltpu.VMEM((S_G, 128), jnp.float32),
            pltpu.VMEM((ACC_R, HD // 2), jnp.float32),
            pltpu.VMEM((ACC_R * 2 * H,), jnp.float32),
            pltpu.SemaphoreType.DMA,
            pltpu.SemaphoreType.DMA,
        ],
    )
    return f(m2f, s2, dst)


@jax.jit
def _run(v, e, edge_index, Wq, bq, Wk, bk, Wv, bv, We, be):
    wT = jnp.concatenate([Wq, Wk, Wv], axis=0).T
    b = jnp.concatenate([bq, bk, bv])[None, :]
    qkv = _qkv_call(v, wT, b)
    q_t = qkv[:, :HD]
    k_t = qkv[:, HD:2 * HD]
    v_t = qkv[:, 2 * HD:]
    src = edge_index[0]
    dst = edge_index[1]
    ks, qd, vs = _gather_call(k_t, q_t, v_t, src, dst)
    e_out, m2, s2 = _edge_call(e, ks, qd, vs, We.T, be[None, :])
    v_pad = _scatter_call(m2, s2, dst)
    return v_pad[:N].reshape(N, H, D), e_out.reshape(E, H, D)


def kernel(v, e, edge_index, Wq, bq, Wk, bk, Wv, bv, We, be):
    return _run(v, e, edge_index, Wq, bq, Wk, bk, Wv, bv, We, be)


# speedup vs baseline: 12.8230x; 1.0882x over previous
import functools

import numpy as np
import jax
import jax.numpy as jnp
from jax import lax
from jax.experimental import pallas as pl
from jax.experimental.pallas import tpu as pltpu
from jax.experimental.pallas import tpu_sc as plsc

N = 10000
E = 160000
IN_DIM = 256
H = 8
D = 32
HD = H * D

NC, NS = 2, 16
NW = NC * NS
EPW = E // NW
G_CH = 200
S_CH = 640
S_G = 64
S_CB = 640
S_TR = S_CB + S_G + 32
NRN = 632
VROWS = (NW // 2) * NRN
DUMMY = NRN
ACC_R = NRN + 8

_SCALE = np.float32(1.0 / np.sqrt(np.float32(D)))


def _mesh():
    return plsc.VectorSubcoreMesh(
        core_axis_name="c", subcore_axis_name="s", num_cores=NC, num_subcores=NS
    )


def _qkv_body(v_ref, w_ref, b_ref, o_ref):
    o_ref[...] = (
        jnp.dot(v_ref[...], w_ref[...], preferred_element_type=jnp.float32)
        + b_ref[...]
    )


def _qkv_call(v, wT, b):
    return pl.pallas_call(
        _qkv_body,
        out_shape=jax.ShapeDtypeStruct((N, 3 * HD), jnp.float32),
    )(v, wT, b)


def _gather_body(kt, qt, vt, srci, dsti, kout, qout, vout, idxs_v, idxd_v,
                 rows_v, sem):
    c = lax.axis_index("c")
    s = lax.axis_index("s")
    wid = s * NC + c
    base = wid * EPW
    pltpu.sync_copy(srci.at[pl.ds(base, EPW)], idxs_v)
    pltpu.sync_copy(dsti.at[pl.ds(base, EPW)], idxd_v)

    def step(i, carry):
        off = i * G_CH
        sl = pl.ds(off, G_CH)
        out_sl = pl.ds(base + off, G_CH)
        pltpu.async_copy(kt.at[idxs_v.at[sl]], rows_v, sem).wait()
        pltpu.sync_copy(rows_v, kout.at[out_sl])
        pltpu.async_copy(qt.at[idxd_v.at[sl]], rows_v, sem).wait()
        pltpu.sync_copy(rows_v, qout.at[out_sl])
        pltpu.async_copy(vt.at[idxs_v.at[sl]], rows_v, sem).wait()
        pltpu.sync_copy(rows_v, vout.at[out_sl])
        return carry

    lax.fori_loop(0, EPW // G_CH, step, 0)


def _gather_call(k_t, q_t, v_t, src, dst):
    f = pl.kernel(
        _gather_body,
        out_type=[jax.ShapeDtypeStruct((E, HD), jnp.float32)] * 3,
        mesh=_mesh(),
        scratch_types=[
            pltpu.VMEM((EPW,), jnp.int32),
            pltpu.VMEM((EPW,), jnp.int32),
            pltpu.VMEM((G_CH, HD), jnp.float32),
            pltpu.SemaphoreType.DMA,
        ],
    )
    return f(k_t, q_t, v_t, src, dst)


def _edge_body(e_ref, ks_ref, qd_ref, vs_ref, weT_ref, be_ref,
               eout_ref, msg_ref, s2_ref):
    p = (
        jnp.dot(e_ref[...], weT_ref[...], preferred_element_type=jnp.float32)
        + be_ref[...]
    )
    score = ks_ref[...] * qd_ref[...] * _SCALE * p
    eout_ref[...] = score
    hsel = (
        lax.broadcasted_iota(jnp.int32, (HD, H), 0) // D
        == lax.broadcasted_iota(jnp.int32, (HD, H), 1)
    ).astype(jnp.float32)
    hselT = (
        lax.broadcasted_iota(jnp.int32, (H, HD), 1) // D
        == lax.broadcasted_iota(jnp.int32, (H, HD), 0)
    ).astype(jnp.float32)
    t = jnp.dot(score, hsel, preferred_element_type=jnp.float32)
    s = jnp.exp(jnp.clip(t, -5.0, 5.0))
    sb = jnp.dot(s, hselT, preferred_element_type=jnp.float32)
    msg = vs_ref[...] * sb
    msg_ref[0] = msg[:, :HD // 2]
    msg_ref[1] = msg[:, HD // 2:]
    s2 = jnp.concatenate([s, s], axis=1)
    er = lax.broadcasted_iota(jnp.int32, (EB, 128), 0)
    cc = lax.broadcasted_iota(jnp.int32, (EB, 128), 1)
    s2w = jnp.tile(s2, (1, 8)) * (er % 8 == cc // 16).astype(jnp.float32)
    br = lax.broadcasted_iota(jnp.int32, (EB // 8, EB), 0)
    be = lax.broadcasted_iota(jnp.int32, (EB // 8, EB), 1)
    bsel = (be // 8 == br).astype(jnp.float32)
    s2_ref[...] = jnp.dot(bsel, s2w, preferred_element_type=jnp.float32)


EB = 1600


def _edge_call(e, ks, qd, vs, weT, be):
    grid = (E // EB,)
    bs_e = pl.BlockSpec((EB, IN_DIM), lambda i: (i, 0))
    bs_hd = pl.BlockSpec((EB, HD), lambda i: (i, 0))
    return pl.pallas_call(
        _edge_body,
        grid=grid,
        in_specs=[
            bs_e, bs_hd, bs_hd, bs_hd,
            pl.BlockSpec((IN_DIM, HD), lambda i: (0, 0)),
            pl.BlockSpec((1, HD), lambda i: (0, 0)),
        ],
        out_specs=[
            bs_hd,
            pl.BlockSpec((2, EB, HD // 2), lambda i: (0, i, 0)),
            pl.BlockSpec((EB // 8, 128), lambda i: (i, 0)),
        ],
        out_shape=[
            jax.ShapeDtypeStruct((E, HD), jnp.float32),
            jax.ShapeDtypeStruct((2, E, HD // 2), jnp.float32),
            jax.ShapeDtypeStruct((E // 8, 128), jnp.float32),
        ],
    )(e, ks, qd, vs, weT, be)


def _vgather(vec, idx):
    dnums = lax.GatherDimensionNumbers(
        offset_dims=(), collapsed_slice_dims=(0,), start_index_map=(0,)
    )
    return lax.gather(
        vec, idx[:, None], dnums, slice_sizes=(1,),
        mode=lax.GatherScatterMode.PROMISE_IN_BOUNDS,
    )


def _scatter_body(m2f, s2p, dsti, vout,
                  idx2, cbl2, cbe2, idx82, idxm2, idx8e, idxme,
                  gbuf, sgbuf, acc, accz, dsem, gsem):
    c = lax.axis_index("c")
    sid = lax.axis_index("s")
    widx = c * NS + sid
    nr = widx // 2
    ch = widx % 2
    base = nr * NRN
    NCH = E // S_CH

    iota16 = lax.broadcasted_iota(jnp.int32, (16,), 0)
    zeros16 = iota16 * 0
    zf16 = zeros16.astype(jnp.float32)

    def zinit(rr, c2):
        for k in range(8):
            acc[rr, pl.ds(k * 16, 16)] = zf16
        accz[pl.ds(rr * 16, 16)] = zf16
        return c2

    lax.fori_loop(0, ACC_R, zinit, 0)

    def scan(bank, off):
        cb = bank * S_TR

        def sgrp(g, pos):
            dstv = idx2[pl.ds(bank * S_CH + g * 16, 16)]
            l = dstv - base
            inr = (l >= 0) & (l < NRN)
            gid = iota16 + (off + g * 16)
            inri = inr.astype(jnp.int32)
            csum = plsc.cumsum(inri)
            tgt = cb + jnp.where(inr, pos + (csum - inri), S_CB + iota16)
            plsc.store_scatter(cbl2, [tgt], jnp.where(inr, l, DUMMY))
            plsc.store_scatter(cbe2, [tgt], jnp.where(inr, gid, 0))
            return pos + csum[15]

        pos = lax.fori_loop(0, S_CH // 16, sgrp, 0)
        dummy16 = zeros16 + DUMMY

        def pre(k, c2):
            cbl2[pl.ds(cb + pos + k * 16, 16)] = dummy16
            return c2

        lax.fori_loop(0, (S_G // 16) + 1, pre, 0)
        return pos

    def fire(bank):
        cb = bank * S_TR

        def i8(g, c2):
            ev = cbe2[pl.ds(cb + g * 16, 16)]
            idx82[pl.ds(bank * S_G + g * 16, 16)] = ev // 8
            idxm2[pl.ds(bank * S_G + g * 16, 16)] = ev + ch * E
            return c2

        lax.fori_loop(0, S_G // 16, i8, 0)
        pltpu.async_copy(m2f.at[idxm2.at[pl.ds(bank * S_G, S_G)]], gbuf, gsem)
        pltpu.async_copy(s2p.at[idx82.at[pl.ds(bank * S_G, S_G)]], sgbuf,
                         gsem)

    def drain_gathers():
        pltpu.make_async_copy(m2f.at[pl.ds(0, S_G)], gbuf, gsem).wait()
        pltpu.make_async_copy(m2f.at[pl.ds(0, S_G)], sgbuf, gsem).wait()

    def process_block(bank, rb, ngrp=S_G // 16):
        cb = bank * S_TR

        def one(jg, c3):
            lv = cbl2[pl.ds(cb + rb + jg * 16, 16)]
            ev = cbe2[pl.ds(cb + rb + jg * 16, 16)]
            for jj in range(16):
                lrow = lv[jj]
                j = jg * 16 + jj
                for k in range(8):
                    plsc.addupdate(acc.at[lrow, pl.ds(k * 16, 16)],
                                   gbuf[j, pl.ds(k * 16, 16)])
                q = lax.rem(ev[jj], 8)
                plsc.addupdate(accz.at[pl.ds(lrow * 16, 16)],
                               sgbuf[j, pl.ds(q * 16, 16)])
            return c3

        lax.fori_loop(0, ngrp, one, 0)

    def extra_rounds(bank, pos):
        cb = bank * S_TR
        nrnd = (pos + S_G - 1) // S_G

        @pl.when(nrnd > 1)
        def _():
            def rnd(r, c2):
                rb = pl.multiple_of(r * S_G, S_G)

                def i8(g, c3):
                    ev = cbe2[pl.ds(cb + rb + g * 16, 16)]
                    idx8e[pl.ds(g * 16, 16)] = ev // 8
                    idxme[pl.ds(g * 16, 16)] = ev + ch * E
                    return c3

                lax.fori_loop(0, S_G // 16, i8, 0)
                pltpu.async_copy(m2f.at[idxme], gbuf, gsem).wait()
                pltpu.async_copy(s2p.at[idx8e], sgbuf, gsem).wait()
                process_block(bank, rb)
                return c2

            lax.fori_loop(1, nrnd, rnd, 0)

    pltpu.sync_copy(dsti.at[pl.ds(0, S_CH)], idx2.at[pl.ds(0, S_CH)])
    pos0 = scan(0, 0)
    fire(0)
    pltpu.async_copy(dsti.at[pl.ds(S_CH, S_CH)], idx2.at[pl.ds(S_CH, S_CH)],
                     dsem)

    def step(i, pos_i):
        bank = lax.rem(i, 2)
        nbank = 1 - bank
        pltpu.make_async_copy(dsti.at[pl.ds(0, S_CH)],
                              idx2.at[pl.ds(nbank * S_CH, S_CH)], dsem).wait()
        nxt = jnp.minimum(i + 1, NCH - 1)
        pos_n = scan(nbank, nxt * S_CH)

        @pl.when(i < NCH - 1)
        def _():
            nxt2 = jnp.minimum(i + 2, NCH - 1)
            pltpu.async_copy(dsti.at[pl.ds(nxt2 * S_CH, S_CH)],
                             idx2.at[pl.ds(bank * S_CH, S_CH)], dsem)

        drain_gathers()
        process_block(bank, 0, jnp.minimum((pos_i + 15) // 16, S_G // 16))
        extra_rounds(bank, pos_i)

        @pl.when(i + 1 < NCH)
        def _():
            fire(nbank)

        return pos_n

    lax.fori_loop(0, NCH, step, pos0)

    def norm(rr, c2):
        zrow = accz[pl.ds(rr * 16, 16)]
        for k in range(8):
            hk = 4 * ch + k // 2
            zs = _vgather(zrow, zeros16 + hk)
            vals = acc[rr, pl.ds(k * 16, 16)]
            acc[rr, pl.ds(k * 16, 16)] = vals / (zs + 1e-6)
        return c2

    lax.fori_loop(0, NRN, norm, 0)

    def wb(k, c2):
        pltpu.sync_copy(
            acc.at[pl.ds(k * 8, 8), pl.ds(0, HD // 2)],
            vout.at[pl.ds(base + k * 8, 8), pl.ds(ch * (HD // 2), HD // 2)])
        return c2

    lax.fori_loop(0, NRN // 8, wb, 0)


def _scatter_call(m2, s2, dst):
    m2f = m2.reshape(2 * E, HD // 2)
    f = pl.kernel(
        _scatter_body,
        out_type=jax.ShapeDtypeStruct((VROWS, HD), jnp.float32),
        mesh=_mesh(),
        compiler_params=pltpu.CompilerParams(
            needs_layout_passes=False,
            internal_scratch_in_bytes=1 << 20,
        ),
        scratch_types=[
            pltpu.VMEM((2 * S_CH,), jnp.int32),
            pltpu.VMEM((2 * S_TR,), jnp.int32),
            pltpu.VMEM((2 * S_TR,), jnp.int32),
            pltpu.VMEM((2 * S_G,), jnp.int32),
            pltpu.VMEM((2 * S_G,), jnp.int32),
            pltpu.VMEM((S_G,), jnp.int32),
            pltpu.VMEM((S_G,), jnp.int32),
            pltpu.VMEM((S_G, HD // 2), jnp.float32),
            pltpu.VMEM((S_G, 128), jnp.float32),
            pltpu.VMEM((ACC_R, HD // 2), jnp.float32),
            pltpu.VMEM((ACC_R * 2 * H,), jnp.float32),
            pltpu.SemaphoreType.DMA,
            pltpu.SemaphoreType.DMA,
        ],
    )
    return f(m2f, s2, dst)


@jax.jit
def _run(v, e, edge_index, Wq, bq, Wk, bk, Wv, bv, We, be):
    wT = jnp.concatenate([Wq, Wk, Wv], axis=0).T
    b = jnp.concatenate([bq, bk, bv])[None, :]
    qkv = _qkv_call(v, wT, b)
    q_t = qkv[:, :HD]
    k_t = qkv[:, HD:2 * HD]
    v_t = qkv[:, 2 * HD:]
    src = edge_index[0]
    dst = edge_index[1]
    ks, qd, vs = _gather_call(k_t, q_t, v_t, src, dst)
    e_out, m2, s2 = _edge_call(e, ks, qd, vs, We.T, be[None, :])
    v_pad = _scatter_call(m2, s2, dst)
    return v_pad[:N].reshape(N, H, D), e_out.reshape(E, H, D)


def kernel(v, e, edge_index, Wq, bq, Wk, bk, Wv, bv, We, be):
    return _run(v, e, edge_index, Wq, bq, Wk, bk, Wv, bv, We, be)
